# baseline (speedup 1.0000x reference)
import functools
import math

import jax
import jax.numpy as jnp
from jax import lax
from jax.experimental import pallas as pl
from jax.experimental.pallas import tpu as pltpu
from jax.experimental.pallas import tpu_sc as plsc

L = 16
NS = 16
NC = 2
CH = 128
NCH = 4
STEP = NCH * CH

_SC_PARAMS = pltpu.CompilerParams(
    needs_layout_passes=False, use_tc_tiling_on_sc=False)


def _fc1_body(x_ref, w_ref, b_ref, o_ref):
    h = lax.dot_general(x_ref[...], w_ref[...], (((0,), (1,)), ((), ())),
                        preferred_element_type=jnp.float32)
    o_ref[...] = h + b_ref[...]


def _fc1(x2, W1, b1, nb):
    in_c, Np = x2.shape
    hid = W1.shape[0]
    Nb = Np // nb
    return pl.pallas_call(
        _fc1_body,
        grid=(nb,),
        in_specs=[
            pl.BlockSpec((in_c, Nb), lambda i: (0, i)),
            pl.BlockSpec((hid, in_c), lambda i: (0, 0)),
            pl.BlockSpec((1, hid), lambda i: (0, 0)),
        ],
        out_specs=pl.BlockSpec((Nb, hid), lambda i: (i, 0)),
        out_shape=jax.ShapeDtypeStruct((Np, hid), jnp.float32),
    )(x2, W1, b1.reshape(1, hid))


def _fc2_body(a_ref, w_ref, b_ref, o_ref):
    o = lax.dot_general(w_ref[...], a_ref[...], (((1,), (1,)), ((), ())),
                        preferred_element_type=jnp.float32)
    o_ref[...] = o + b_ref[...]


def _fc2(a2, W2, b2, nb):
    Np, hid = a2.shape
    out_c = W2.shape[0]
    Nb = Np // nb
    return pl.pallas_call(
        _fc2_body,
        grid=(nb,),
        in_specs=[
            pl.BlockSpec((Nb, hid), lambda i: (i, 0)),
            pl.BlockSpec((out_c, hid), lambda i: (0, 0)),
            pl.BlockSpec((out_c, 1), lambda i: (0, 0)),
        ],
        out_specs=pl.BlockSpec((out_c, Nb), lambda i: (0, i)),
        out_shape=jax.ShapeDtypeStruct((out_c, Np), jnp.float32),
    )(a2, W2, b2.reshape(out_c, 1))


def _sc_weights(es2d, ed2d, p2, prm):
    R = es2d.shape[0]
    RW = R // (NC * NS)
    M = RW // NCH
    Q = (M + 2 + 3) // 4 + 1
    Ep = R * CH

    mesh = plsc.VectorSubcoreMesh(core_axis_name="c", subcore_axis_name="s")

    @functools.partial(
        pl.kernel,
        out_type=jax.ShapeDtypeStruct((Ep,), jnp.float32),
        mesh=mesh,
        compiler_params=_SC_PARAMS,
        scratch_types=(
            [pltpu.VMEM((NCH, CH), jnp.int32) for _ in range(4)]
            + [pltpu.VMEM((NCH, CH), jnp.int32) for _ in range(4)]
            + [pltpu.VMEM((STEP, 16), jnp.float32) for _ in range(2)]
            + [pltpu.VMEM((STEP, 16), jnp.float32) for _ in range(2)]
            + [pltpu.VMEM((STEP,), jnp.float32) for _ in range(2)]
            + [pltpu.VMEM((32,), jnp.float32)]
            + [pltpu.SemaphoreType.DMA for _ in range(8)]
        ),
    )
    def k(es_hbm, ed_hbm, p_hbm, prm_hbm, w_hbm, *scr):
        si = scr[0:4]
        di = scr[4:8]
        ps = scr[8:10]
        pd = scr[10:12]
        wo = scr[12:14]
        prm_v = scr[14]
        sem_i = scr[15:19]
        sem_g = scr[19:21]
        sem_o = scr[21:23]

        cid = lax.axis_index("c")
        sid = lax.axis_index("s")
        wid = cid * NS + sid
        row0 = wid * RW

        pltpu.sync_copy(prm_hbm, prm_v)
        nbw = prm_v[pl.ds(0, L)]
        cmul = prm_v[pl.ds(L, L)]

        zc = jnp.zeros((L,), jnp.int32)
        c1 = jnp.full((L,), 1, jnp.int32)
        c2 = jnp.full((L,), 2, jnp.int32)
        c3 = jnp.full((L,), 3, jnp.int32)

        def fire_idx(m, s):
            r0 = row0 + m * NCH
            pltpu.async_copy(es_hbm.at[pl.ds(r0, NCH)], si[s], sem_i[s])
            pltpu.async_copy(ed_hbm.at[pl.ds(r0, NCH)], di[s], sem_i[s])

        def wait_idx(m, s):
            r0 = row0 + m * NCH
            pltpu.make_async_copy(es_hbm.at[pl.ds(r0, NCH)], si[s], sem_i[s]).wait()
            pltpu.make_async_copy(ed_hbm.at[pl.ds(r0, NCH)], di[s], sem_i[s]).wait()

        fire_idx(0, 0)
        fire_idx(1, 1)

        def outer(q, carry):
            for b in range(4):
                m = q * 4 + b
                s = b
                s2 = (b + 2) % 4
                sp = (b + 3) % 4
                d = b % 2
                dp = (b + 1) % 2

                @pl.when(jnp.logical_and(m >= 2, m <= M + 1))
                def _():
                    e0 = row0 * CH + (m - 2) * STEP
                    pltpu.make_async_copy(
                        wo[d], w_hbm.at[pl.ds(e0, STEP)], sem_o[d]).wait()

                @pl.when(m + 2 <= M - 1)
                def _():
                    fire_idx(m + 2, s2)

                @pl.when(m <= M - 1)
                def _():
                    wait_idx(m, s)
                    for j in range(NCH):
                        pltpu.async_copy(p_hbm.at[si[s].at[j]],
                                         ps[d].at[pl.ds(j * CH, CH)], sem_g[d])
                        pltpu.async_copy(p_hbm.at[di[s].at[j]],
                                         pd[d].at[pl.ds(j * CH, CH)], sem_g[d])

                @pl.when(jnp.logical_and(m >= 1, m <= M))
                def _():
                    for j in range(NCH):
                        pltpu.make_async_copy(
                            p_hbm.at[si[sp].at[j]],
                            ps[dp].at[pl.ds(j * CH, CH)], sem_g[dp]).wait()
                        pltpu.make_async_copy(
                            p_hbm.at[di[sp].at[j]],
                            pd[dp].at[pl.ds(j * CH, CH)], sem_g[dp]).wait()

                    def grp(g, c):
                        rows = g * L + lax.iota(jnp.int32, L)
                        sx = plsc.load_gather(ps[dp], [rows, zc])
                        sy = plsc.load_gather(ps[dp], [rows, c1])
                        sz = plsc.load_gather(ps[dp], [rows, c2])
                        sw = plsc.load_gather(ps[dp], [rows, c3])
                        dx = plsc.load_gather(pd[dp], [rows, zc])
                        dy = plsc.load_gather(pd[dp], [rows, c1])
                        dz = plsc.load_gather(pd[dp], [rows, c2])
                        ex = sx - dx
                        ey = sy - dy
                        ez = sz - dz
                        d2 = ex * ex + ey * ey + ez * ez
                        wo[dp][pl.ds(g * L, L)] = cmul * jnp.exp(nbw * d2) * sw
                        return c
                    lax.fori_loop(0, STEP // L, grp, 0)
                    e0 = row0 * CH + (m - 1) * STEP
                    pltpu.async_copy(wo[dp], w_hbm.at[pl.ds(e0, STEP)], sem_o[dp])
            return carry
        lax.fori_loop(0, Q, outer, 0)

    return k(es2d, ed2d, p2, prm)


def _sc_scatter(es2d, ed2d, wts, hcat, N):
    R = es2d.shape[0]
    RT = R // NS
    M = RT // NCH
    Q = (M + 2 + 3) // 4 + 1
    ACC_R = ((N + 1 + NS * CH - 1) // (NS * CH)) * NS * CH
    ZCH = ACC_R // (NS * CH)
    Nup = ((N + NS * 8 - 1) // (NS * 8)) * (NS * 8)
    ORT = Nup // NS

    mesh = plsc.VectorSubcoreMesh(core_axis_name="c", subcore_axis_name="s")

    @functools.partial(
        pl.kernel,
        out_type=jax.ShapeDtypeStruct((2 * Nup, 16), jnp.float32),
        mesh=mesh,
        compiler_params=_SC_PARAMS,
        scratch_types=(
            [pltpu.VMEM((NCH, CH), jnp.int32) for _ in range(4)]
            + [pltpu.VMEM((NCH, CH), jnp.int32) for _ in range(4)]
            + [pltpu.VMEM((NCH, CH), jnp.int32) for _ in range(4)]
            + [pltpu.VMEM((STEP,), jnp.float32) for _ in range(4)]
            + [pltpu.VMEM((STEP, 16), jnp.float32) for _ in range(2)]
            + [pltpu.VMEM((STEP, 16), jnp.float32) for _ in range(2)]
            + [pltpu.VMEM_SHARED((ACC_R, 16), jnp.float32)]
            + [pltpu.SemaphoreType.DMA for _ in range(8)]
        ),
    )
    def k(es_hbm, ed_hbm, w_hbm, h_hbm, out_hbm, *scr):
        si = scr[0:4]
        di = scr[4:8]
        s2 = scr[8:12]
        wv = scr[12:16]
        hs = scr[16:18]
        ho = scr[18:20]
        acc = scr[20]
        sem_i = scr[21:25]
        sem_g = scr[25:27]
        sem_s = scr[27:29]

        cid = lax.axis_index("c")
        sid = lax.axis_index("s")
        row0 = sid * RT
        cNv = jnp.full((L,), cid * N, jnp.int32)

        def _zrow(r, c):
            hs[0][r, pl.ds(0, L)] = jnp.zeros((L,), jnp.float32)
            return c
        lax.fori_loop(0, CH, _zrow, 0)
        for jz in range(ZCH):
            pltpu.sync_copy(hs[0].at[pl.ds(0, CH)],
                            acc.at[pl.ds(sid * ZCH * CH + jz * CH, CH)])
        plsc.subcore_barrier()

        def fire_idx(m, s):
            r0 = row0 + m * NCH
            pltpu.async_copy(es_hbm.at[pl.ds(r0, NCH)], si[s], sem_i[s])
            pltpu.async_copy(ed_hbm.at[pl.ds(r0, NCH)], di[s], sem_i[s])
            pltpu.async_copy(w_hbm.at[pl.ds(r0 * CH, STEP)], wv[s], sem_i[s])

        def wait_idx(m, s):
            r0 = row0 + m * NCH
            pltpu.make_async_copy(es_hbm.at[pl.ds(r0, NCH)], si[s], sem_i[s]).wait()
            pltpu.make_async_copy(ed_hbm.at[pl.ds(r0, NCH)], di[s], sem_i[s]).wait()
            pltpu.make_async_copy(w_hbm.at[pl.ds(r0 * CH, STEP)], wv[s], sem_i[s]).wait()

        fire_idx(0, 0)
        fire_idx(1, 1)

        def outer(q, carry):
            for b in range(4):
                m = q * 4 + b
                s = b
                sn = (b + 2) % 4
                sp = (b + 3) % 4
                d = b % 2
                dp = (b + 1) % 2

                @pl.when(jnp.logical_and(m >= 2, m <= M + 1))
                def _():
                    for j in range(NCH):
                        pltpu.make_async_copy(
                            ho[d].at[pl.ds(j * CH, CH)],
                            acc.at[di[sn].at[j]], sem_s[d]).wait()

                @pl.when(m + 2 <= M - 1)
                def _():
                    fire_idx(m + 2, sn)

                @pl.when(m <= M - 1)
                def _():
                    wait_idx(m, s)

                    def adds(t, c):
                        j = t // 8
                        col = (t % 8) * L
                        s2[s][j, pl.ds(col, L)] = si[s][j, pl.ds(col, L)] + cNv
                        return c
                    lax.fori_loop(0, NCH * 8, adds, 0)
                    for j in range(NCH):
                        pltpu.async_copy(h_hbm.at[s2[s].at[j]],
                                         hs[d].at[pl.ds(j * CH, CH)], sem_g[d])

                @pl.when(jnp.logical_and(m >= 1, m <= M))
                def _():
                    for j in range(NCH):
                        pltpu.make_async_copy(
                            h_hbm.at[s2[sp].at[j]],
                            hs[dp].at[pl.ds(j * CH, CH)], sem_g[dp]).wait()

                    def grp(g, c):
                        rows = g * L + lax.iota(jnp.int32, L)
                        w16 = wv[sp][pl.ds(g * L, L)]
                        for col in range(16):
                            cv = jnp.full((L,), col, jnp.int32)
                            hv = plsc.load_gather(hs[dp], [rows, cv])
                            plsc.store_scatter(ho[dp], [rows, cv], hv * w16)
                        return c
                    lax.fori_loop(0, STEP // L, grp, 0)
                    for j in range(NCH):
                        pltpu.async_copy(ho[dp].at[pl.ds(j * CH, CH)],
                                         acc.at[di[sp].at[j]], sem_s[dp],
                                         add=True)
            return carry
        lax.fori_loop(0, Q, outer, 0)

        plsc.subcore_barrier()
        pltpu.sync_copy(acc.at[pl.ds(sid * ORT, ORT)],
                        out_hbm.at[pl.ds(cid * Nup + sid * ORT, ORT)])

    return k(es2d, ed2d, wts, hcat)


def kernel(x, grid, grid_weight, edge_src, edge_dst, W1, b1, W2, b2, baseweight):
    bsz, in_c, N = x.shape
    phy_dim = grid.shape[2]
    hid = W1.shape[0]
    out_c = W2.shape[0]
    K = edge_src.shape[2]
    E = bsz * N * K
    HH = hid // (2 * NC)

    NB = 1024
    Np = ((N + NB - 1) // NB) * NB
    nb = Np // NB

    x_p = jnp.pad(x[0], ((0, 0), (0, Np - N)))
    H = _fc1(x_p, W1, b1, nb)

    p2 = jnp.concatenate([grid[0], grid_weight[0][:, None]], axis=1)
    p2 = jnp.pad(p2, ((0, 16), (0, 12)))

    EPT = NC * NS * 8 * CH
    Ep = ((E + EPT - 1) // EPT) * EPT
    es = jnp.concatenate(
        [edge_src.reshape(-1), jnp.zeros((Ep - E,), jnp.int32)]).reshape(-1, CH)
    ed = jnp.concatenate(
        [edge_dst.reshape(-1), jnp.full((Ep - E,), N, jnp.int32)]).reshape(-1, CH)

    bw = baseweight.reshape(())
    cmul = jnp.sqrt((bw / math.pi) ** phy_dim)
    prm = jnp.concatenate([jnp.broadcast_to(-bw, (L,)),
                           jnp.broadcast_to(cmul, (L,))])

    wts = _sc_weights(es, ed, p2, prm)

    hq = [H[:N, q * HH:(q + 1) * HH] for q in range(4)]
    msg0 = _sc_scatter(es, ed, wts, jnp.concatenate([hq[0], hq[1]], axis=0), N)
    msg1 = _sc_scatter(es, ed, wts, jnp.concatenate([hq[2], hq[3]], axis=0), N)
    Nup = ((N + NS * 8 - 1) // (NS * 8)) * (NS * 8)

    a2 = jnp.concatenate([msg0[:N], msg0[Nup:Nup + N],
                          msg1[:N], msg1[Nup:Nup + N]], axis=1)
    a2 = jnp.pad(a2, ((0, Np - N), (0, 0)))
    out = _fc2(a2, W2, b2, nb)
    return out[:, :N].reshape(bsz, out_c, N)


# speedup vs baseline: 1.0016x; 1.0016x over previous
import functools
import math

import jax
import jax.numpy as jnp
from jax import lax
from jax.experimental import pallas as pl
from jax.experimental.pallas import tpu as pltpu
from jax.experimental.pallas import tpu_sc as plsc

L = 16
NS = 16
NC = 2
CH = 128
NCH = 8
STEP = NCH * CH

_SC_PARAMS = pltpu.CompilerParams(
    needs_layout_passes=False, use_tc_tiling_on_sc=False)


def _fc1_body(x_ref, w_ref, b_ref, o_ref):
    h = lax.dot_general(x_ref[...], w_ref[...], (((0,), (1,)), ((), ())),
                        preferred_element_type=jnp.float32)
    o_ref[...] = h + b_ref[...]


def _fc1(x2, W1, b1, nb):
    in_c, Np = x2.shape
    hid = W1.shape[0]
    Nb = Np // nb
    return pl.pallas_call(
        _fc1_body,
        grid=(nb,),
        in_specs=[
            pl.BlockSpec((in_c, Nb), lambda i: (0, i)),
            pl.BlockSpec((hid, in_c), lambda i: (0, 0)),
            pl.BlockSpec((1, hid), lambda i: (0, 0)),
        ],
        out_specs=pl.BlockSpec((Nb, hid), lambda i: (i, 0)),
        out_shape=jax.ShapeDtypeStruct((Np, hid), jnp.float32),
    )(x2, W1, b1.reshape(1, hid))


def _fc2_body(a_ref, w_ref, b_ref, o_ref):
    o = lax.dot_general(w_ref[...], a_ref[...], (((1,), (1,)), ((), ())),
                        preferred_element_type=jnp.float32)
    o_ref[...] = o + b_ref[...]


def _fc2(a2, W2, b2, nb):
    Np, hid = a2.shape
    out_c = W2.shape[0]
    Nb = Np // nb
    return pl.pallas_call(
        _fc2_body,
        grid=(nb,),
        in_specs=[
            pl.BlockSpec((Nb, hid), lambda i: (i, 0)),
            pl.BlockSpec((out_c, hid), lambda i: (0, 0)),
            pl.BlockSpec((out_c, 1), lambda i: (0, 0)),
        ],
        out_specs=pl.BlockSpec((out_c, Nb), lambda i: (0, i)),
        out_shape=jax.ShapeDtypeStruct((out_c, Np), jnp.float32),
    )(a2, W2, b2.reshape(out_c, 1))


def _sc_weights(es2d, ed2d, p2, prm):
    R = es2d.shape[0]
    RW = R // (NC * NS)
    M = RW // NCH
    Q = (M + 2 + 3) // 4 + 1
    Ep = R * CH

    mesh = plsc.VectorSubcoreMesh(core_axis_name="c", subcore_axis_name="s")

    @functools.partial(
        pl.kernel,
        out_type=jax.ShapeDtypeStruct((Ep,), jnp.float32),
        mesh=mesh,
        compiler_params=_SC_PARAMS,
        scratch_types=(
            [pltpu.VMEM((NCH, CH), jnp.int32) for _ in range(4)]
            + [pltpu.VMEM((NCH, CH), jnp.int32) for _ in range(4)]
            + [pltpu.VMEM((STEP, 16), jnp.float32) for _ in range(2)]
            + [pltpu.VMEM((STEP, 16), jnp.float32) for _ in range(2)]
            + [pltpu.VMEM((STEP,), jnp.float32) for _ in range(2)]
            + [pltpu.VMEM((32,), jnp.float32)]
            + [pltpu.SemaphoreType.DMA for _ in range(8)]
        ),
    )
    def k(es_hbm, ed_hbm, p_hbm, prm_hbm, w_hbm, *scr):
        si = scr[0:4]
        di = scr[4:8]
        ps = scr[8:10]
        pd = scr[10:12]
        wo = scr[12:14]
        prm_v = scr[14]
        sem_i = scr[15:19]
        sem_g = scr[19:21]
        sem_o = scr[21:23]

        cid = lax.axis_index("c")
        sid = lax.axis_index("s")
        wid = cid * NS + sid
        row0 = wid * RW

        pltpu.sync_copy(prm_hbm, prm_v)
        nbw = prm_v[pl.ds(0, L)]
        cmul = prm_v[pl.ds(L, L)]

        zc = jnp.zeros((L,), jnp.int32)
        c1 = jnp.full((L,), 1, jnp.int32)
        c2 = jnp.full((L,), 2, jnp.int32)
        c3 = jnp.full((L,), 3, jnp.int32)

        def fire_idx(m, s):
            r0 = row0 + m * NCH
            pltpu.async_copy(es_hbm.at[pl.ds(r0, NCH)], si[s], sem_i[s])
            pltpu.async_copy(ed_hbm.at[pl.ds(r0, NCH)], di[s], sem_i[s])

        def wait_idx(m, s):
            r0 = row0 + m * NCH
            pltpu.make_async_copy(es_hbm.at[pl.ds(r0, NCH)], si[s], sem_i[s]).wait()
            pltpu.make_async_copy(ed_hbm.at[pl.ds(r0, NCH)], di[s], sem_i[s]).wait()

        fire_idx(0, 0)
        fire_idx(1, 1)

        def outer(q, carry):
            for b in range(4):
                m = q * 4 + b
                s = b
                s2 = (b + 2) % 4
                sp = (b + 3) % 4
                d = b % 2
                dp = (b + 1) % 2

                @pl.when(jnp.logical_and(m >= 2, m <= M + 1))
                def _():
                    e0 = row0 * CH + (m - 2) * STEP
                    pltpu.make_async_copy(
                        wo[d], w_hbm.at[pl.ds(e0, STEP)], sem_o[d]).wait()

                @pl.when(m + 2 <= M - 1)
                def _():
                    fire_idx(m + 2, s2)

                @pl.when(m <= M - 1)
                def _():
                    wait_idx(m, s)
                    for j in range(NCH):
                        pltpu.async_copy(p_hbm.at[si[s].at[j]],
                                         ps[d].at[pl.ds(j * CH, CH)], sem_g[d])
                        pltpu.async_copy(p_hbm.at[di[s].at[j]],
                                         pd[d].at[pl.ds(j * CH, CH)], sem_g[d])

                @pl.when(jnp.logical_and(m >= 1, m <= M))
                def _():
                    for j in range(NCH):
                        pltpu.make_async_copy(
                            p_hbm.at[si[sp].at[j]],
                            ps[dp].at[pl.ds(j * CH, CH)], sem_g[dp]).wait()
                        pltpu.make_async_copy(
                            p_hbm.at[di[sp].at[j]],
                            pd[dp].at[pl.ds(j * CH, CH)], sem_g[dp]).wait()

                    def grp(g, c):
                        rows = g * L + lax.iota(jnp.int32, L)
                        sx = plsc.load_gather(ps[dp], [rows, zc])
                        sy = plsc.load_gather(ps[dp], [rows, c1])
                        sz = plsc.load_gather(ps[dp], [rows, c2])
                        sw = plsc.load_gather(ps[dp], [rows, c3])
                        dx = plsc.load_gather(pd[dp], [rows, zc])
                        dy = plsc.load_gather(pd[dp], [rows, c1])
                        dz = plsc.load_gather(pd[dp], [rows, c2])
                        ex = sx - dx
                        ey = sy - dy
                        ez = sz - dz
                        d2 = ex * ex + ey * ey + ez * ez
                        wo[dp][pl.ds(g * L, L)] = cmul * jnp.exp(nbw * d2) * sw
                        return c
                    lax.fori_loop(0, STEP // L, grp, 0)
                    e0 = row0 * CH + (m - 1) * STEP
                    pltpu.async_copy(wo[dp], w_hbm.at[pl.ds(e0, STEP)], sem_o[dp])
            return carry
        lax.fori_loop(0, Q, outer, 0)

    return k(es2d, ed2d, p2, prm)


def _sc_scatter(es2d, ed2d, wts, hcat, N):
    R = es2d.shape[0]
    RT = R // NS
    M = RT // NCH
    Q = (M + 2 + 3) // 4 + 1
    ACC_R = ((N + 1 + NS * CH - 1) // (NS * CH)) * NS * CH
    ZCH = ACC_R // (NS * CH)
    Nup = ((N + NS * 8 - 1) // (NS * 8)) * (NS * 8)
    ORT = Nup // NS

    mesh = plsc.VectorSubcoreMesh(core_axis_name="c", subcore_axis_name="s")

    @functools.partial(
        pl.kernel,
        out_type=jax.ShapeDtypeStruct((2 * Nup, 16), jnp.float32),
        mesh=mesh,
        compiler_params=_SC_PARAMS,
        scratch_types=(
            [pltpu.VMEM((NCH, CH), jnp.int32) for _ in range(4)]
            + [pltpu.VMEM((NCH, CH), jnp.int32) for _ in range(4)]
            + [pltpu.VMEM((NCH, CH), jnp.int32) for _ in range(4)]
            + [pltpu.VMEM((STEP,), jnp.float32) for _ in range(4)]
            + [pltpu.VMEM((STEP, 16), jnp.float32) for _ in range(2)]
            + [pltpu.VMEM_SHARED((ACC_R, 16), jnp.float32)]
            + [pltpu.SemaphoreType.DMA for _ in range(8)]
        ),
    )
    def k(es_hbm, ed_hbm, w_hbm, h_hbm, out_hbm, *scr):
        si = scr[0:4]
        di = scr[4:8]
        s2 = scr[8:12]
        wv = scr[12:16]
        hs = scr[16:18]
        ho = scr[16:18]
        acc = scr[18]
        sem_i = scr[19:23]
        sem_g = scr[23:25]
        sem_s = scr[25:27]

        cid = lax.axis_index("c")
        sid = lax.axis_index("s")
        row0 = sid * RT
        cNv = jnp.full((L,), cid * N, jnp.int32)

        def _zrow(r, c):
            hs[0][r, pl.ds(0, L)] = jnp.zeros((L,), jnp.float32)
            return c
        lax.fori_loop(0, CH, _zrow, 0)
        for jz in range(ZCH):
            pltpu.sync_copy(hs[0].at[pl.ds(0, CH)],
                            acc.at[pl.ds(sid * ZCH * CH + jz * CH, CH)])
        plsc.subcore_barrier()

        def fire_idx(m, s):
            r0 = row0 + m * NCH
            pltpu.async_copy(es_hbm.at[pl.ds(r0, NCH)], si[s], sem_i[s])
            pltpu.async_copy(ed_hbm.at[pl.ds(r0, NCH)], di[s], sem_i[s])
            pltpu.async_copy(w_hbm.at[pl.ds(r0 * CH, STEP)], wv[s], sem_i[s])

        def wait_idx(m, s):
            r0 = row0 + m * NCH
            pltpu.make_async_copy(es_hbm.at[pl.ds(r0, NCH)], si[s], sem_i[s]).wait()
            pltpu.make_async_copy(ed_hbm.at[pl.ds(r0, NCH)], di[s], sem_i[s]).wait()
            pltpu.make_async_copy(w_hbm.at[pl.ds(r0 * CH, STEP)], wv[s], sem_i[s]).wait()

        fire_idx(0, 0)
        fire_idx(1, 1)

        def outer(q, carry):
            for b in range(4):
                m = q * 4 + b
                s = b
                sn = (b + 2) % 4
                sp = (b + 3) % 4
                d = b % 2
                dp = (b + 1) % 2

                @pl.when(jnp.logical_and(m >= 2, m <= M + 1))
                def _():
                    for j in range(NCH):
                        pltpu.make_async_copy(
                            ho[d].at[pl.ds(j * CH, CH)],
                            acc.at[di[sn].at[j]], sem_s[d]).wait()

                @pl.when(m + 2 <= M - 1)
                def _():
                    fire_idx(m + 2, sn)

                @pl.when(m <= M - 1)
                def _():
                    wait_idx(m, s)

                    def adds(t, c):
                        j = t // 8
                        col = (t % 8) * L
                        s2[s][j, pl.ds(col, L)] = si[s][j, pl.ds(col, L)] + cNv
                        return c
                    lax.fori_loop(0, NCH * 8, adds, 0)
                    for j in range(NCH):
                        pltpu.async_copy(h_hbm.at[s2[s].at[j]],
                                         hs[d].at[pl.ds(j * CH, CH)], sem_g[d])

                @pl.when(jnp.logical_and(m >= 1, m <= M))
                def _():
                    for j in range(NCH):
                        pltpu.make_async_copy(
                            h_hbm.at[s2[sp].at[j]],
                            hs[dp].at[pl.ds(j * CH, CH)], sem_g[dp]).wait()

                    def grp(g, c):
                        rows = g * L + lax.iota(jnp.int32, L)
                        w16 = wv[sp][pl.ds(g * L, L)]
                        for col in range(16):
                            cv = jnp.full((L,), col, jnp.int32)
                            hv = plsc.load_gather(hs[dp], [rows, cv])
                            plsc.store_scatter(ho[dp], [rows, cv], hv * w16)
                        return c
                    lax.fori_loop(0, STEP // L, grp, 0)
                    for j in range(NCH):
                        pltpu.async_copy(ho[dp].at[pl.ds(j * CH, CH)],
                                         acc.at[di[sp].at[j]], sem_s[dp],
                                         add=True)
            return carry
        lax.fori_loop(0, Q, outer, 0)

        plsc.subcore_barrier()
        pltpu.sync_copy(acc.at[pl.ds(sid * ORT, ORT)],
                        out_hbm.at[pl.ds(cid * Nup + sid * ORT, ORT)])

    return k(es2d, ed2d, wts, hcat)


def kernel(x, grid, grid_weight, edge_src, edge_dst, W1, b1, W2, b2, baseweight):
    bsz, in_c, N = x.shape
    phy_dim = grid.shape[2]
    hid = W1.shape[0]
    out_c = W2.shape[0]
    K = edge_src.shape[2]
    E = bsz * N * K
    HH = hid // (2 * NC)

    NB = 1024
    Np = ((N + NB - 1) // NB) * NB
    nb = Np // NB

    x_p = jnp.pad(x[0], ((0, 0), (0, Np - N)))
    H = _fc1(x_p, W1, b1, nb)

    p2 = jnp.concatenate([grid[0], grid_weight[0][:, None]], axis=1)
    p2 = jnp.pad(p2, ((0, 16), (0, 12)))

    EPT = NC * NS * 8 * CH
    Ep = ((E + EPT - 1) // EPT) * EPT
    es = jnp.concatenate(
        [edge_src.reshape(-1), jnp.zeros((Ep - E,), jnp.int32)]).reshape(-1, CH)
    ed = jnp.concatenate(
        [edge_dst.reshape(-1), jnp.full((Ep - E,), N, jnp.int32)]).reshape(-1, CH)

    bw = baseweight.reshape(())
    cmul = jnp.sqrt((bw / math.pi) ** phy_dim)
    prm = jnp.concatenate([jnp.broadcast_to(-bw, (L,)),
                           jnp.broadcast_to(cmul, (L,))])

    wts = _sc_weights(es, ed, p2, prm)

    hq = [H[:N, q * HH:(q + 1) * HH] for q in range(4)]
    msg0 = _sc_scatter(es, ed, wts, jnp.concatenate([hq[0], hq[1]], axis=0), N)
    msg1 = _sc_scatter(es, ed, wts, jnp.concatenate([hq[2], hq[3]], axis=0), N)
    Nup = ((N + NS * 8 - 1) // (NS * 8)) * (NS * 8)

    a2 = jnp.concatenate([msg0[:N], msg0[Nup:Nup + N],
                          msg1[:N], msg1[Nup:Nup + N]], axis=1)
    a2 = jnp.pad(a2, ((0, Np - N), (0, 0)))
    out = _fc2(a2, W2, b2, nb)
    return out[:, :N].reshape(bsz, out_c, N)


# speedup vs baseline: 2.1989x; 2.1954x over previous
import functools
import math

import jax
import jax.numpy as jnp
from jax import lax
from jax.experimental import pallas as pl
from jax.experimental.pallas import tpu as pltpu
from jax.experimental.pallas import tpu_sc as plsc

L = 16
NS = 16
NC = 2
CH = 128
NCH = 8
STEP = NCH * CH

_SC_PARAMS = pltpu.CompilerParams(
    needs_layout_passes=False, use_tc_tiling_on_sc=False)


def _fc1_body(x_ref, w_ref, b_ref, o_ref):
    h = lax.dot_general(x_ref[...], w_ref[...], (((0,), (1,)), ((), ())),
                        preferred_element_type=jnp.float32)
    o_ref[...] = h + b_ref[...]


def _fc1(x2, W1, b1, nb):
    in_c, Np = x2.shape
    hid = W1.shape[0]
    Nb = Np // nb
    return pl.pallas_call(
        _fc1_body,
        grid=(nb,),
        in_specs=[
            pl.BlockSpec((in_c, Nb), lambda i: (0, i)),
            pl.BlockSpec((hid, in_c), lambda i: (0, 0)),
            pl.BlockSpec((1, hid), lambda i: (0, 0)),
        ],
        out_specs=pl.BlockSpec((Nb, hid), lambda i: (i, 0)),
        out_shape=jax.ShapeDtypeStruct((Np, hid), jnp.float32),
    )(x2, W1, b1.reshape(1, hid))


def _fc2_body(a_ref, w_ref, b_ref, o_ref):
    o = lax.dot_general(w_ref[...], a_ref[...], (((1,), (1,)), ((), ())),
                        preferred_element_type=jnp.float32)
    o_ref[...] = o + b_ref[...]


def _fc2(a2, W2, b2, nb):
    Np, hid = a2.shape
    out_c = W2.shape[0]
    Nb = Np // nb
    return pl.pallas_call(
        _fc2_body,
        grid=(nb,),
        in_specs=[
            pl.BlockSpec((Nb, hid), lambda i: (i, 0)),
            pl.BlockSpec((out_c, hid), lambda i: (0, 0)),
            pl.BlockSpec((out_c, 1), lambda i: (0, 0)),
        ],
        out_specs=pl.BlockSpec((out_c, Nb), lambda i: (0, i)),
        out_shape=jax.ShapeDtypeStruct((out_c, Np), jnp.float32),
    )(a2, W2, b2.reshape(out_c, 1))


def _sc_weights(es2d, ed2d, p2, prm):
    R = es2d.shape[0]
    RW = R // (NC * NS)
    M = RW // NCH
    Q = (M + 2 + 3) // 4 + 1
    Ep = R * CH

    mesh = plsc.VectorSubcoreMesh(core_axis_name="c", subcore_axis_name="s")

    @functools.partial(
        pl.kernel,
        out_type=jax.ShapeDtypeStruct((Ep,), jnp.float32),
        mesh=mesh,
        compiler_params=_SC_PARAMS,
        scratch_types=(
            [pltpu.VMEM((NCH, CH), jnp.int32) for _ in range(4)]
            + [pltpu.VMEM((NCH, CH), jnp.int32) for _ in range(4)]
            + [pltpu.VMEM((STEP, 16), jnp.float32) for _ in range(2)]
            + [pltpu.VMEM((STEP, 16), jnp.float32) for _ in range(2)]
            + [pltpu.VMEM((STEP,), jnp.float32) for _ in range(2)]
            + [pltpu.VMEM((32,), jnp.float32)]
            + [pltpu.SemaphoreType.DMA for _ in range(8)]
        ),
    )
    def k(es_hbm, ed_hbm, p_hbm, prm_hbm, w_hbm, *scr):
        si = scr[0:4]
        di = scr[4:8]
        ps = scr[8:10]
        pd = scr[10:12]
        wo = scr[12:14]
        prm_v = scr[14]
        sem_i = scr[15:19]
        sem_g = scr[19:21]
        sem_o = scr[21:23]

        cid = lax.axis_index("c")
        sid = lax.axis_index("s")
        wid = cid * NS + sid
        row0 = wid * RW

        pltpu.sync_copy(prm_hbm, prm_v)
        nbw = prm_v[pl.ds(0, L)]
        cmul = prm_v[pl.ds(L, L)]

        zc = jnp.zeros((L,), jnp.int32)
        c1 = jnp.full((L,), 1, jnp.int32)
        c2 = jnp.full((L,), 2, jnp.int32)
        c3 = jnp.full((L,), 3, jnp.int32)

        def fire_idx(m, s):
            r0 = row0 + m * NCH
            pltpu.async_copy(es_hbm.at[pl.ds(r0, NCH)], si[s], sem_i[s])
            pltpu.async_copy(ed_hbm.at[pl.ds(r0, NCH)], di[s], sem_i[s])

        def wait_idx(m, s):
            r0 = row0 + m * NCH
            pltpu.make_async_copy(es_hbm.at[pl.ds(r0, NCH)], si[s], sem_i[s]).wait()
            pltpu.make_async_copy(ed_hbm.at[pl.ds(r0, NCH)], di[s], sem_i[s]).wait()

        fire_idx(0, 0)
        fire_idx(1, 1)

        def outer(q, carry):
            for b in range(4):
                m = q * 4 + b
                s = b
                s2 = (b + 2) % 4
                sp = (b + 3) % 4
                d = b % 2
                dp = (b + 1) % 2

                @pl.when(jnp.logical_and(m >= 2, m <= M + 1))
                def _():
                    e0 = row0 * CH + (m - 2) * STEP
                    pltpu.make_async_copy(
                        wo[d], w_hbm.at[pl.ds(e0, STEP)], sem_o[d]).wait()

                @pl.when(m + 2 <= M - 1)
                def _():
                    fire_idx(m + 2, s2)

                @pl.when(m <= M - 1)
                def _():
                    wait_idx(m, s)
                    for j in range(NCH):
                        pltpu.async_copy(p_hbm.at[si[s].at[j]],
                                         ps[d].at[pl.ds(j * CH, CH)], sem_g[d])
                        pltpu.async_copy(p_hbm.at[di[s].at[j]],
                                         pd[d].at[pl.ds(j * CH, CH)], sem_g[d])

                @pl.when(jnp.logical_and(m >= 1, m <= M))
                def _():
                    for j in range(NCH):
                        pltpu.make_async_copy(
                            p_hbm.at[si[sp].at[j]],
                            ps[dp].at[pl.ds(j * CH, CH)], sem_g[dp]).wait()
                        pltpu.make_async_copy(
                            p_hbm.at[di[sp].at[j]],
                            pd[dp].at[pl.ds(j * CH, CH)], sem_g[dp]).wait()

                    def grp(g, c):
                        rows = g * L + lax.iota(jnp.int32, L)
                        sx = plsc.load_gather(ps[dp], [rows, zc])
                        sy = plsc.load_gather(ps[dp], [rows, c1])
                        sz = plsc.load_gather(ps[dp], [rows, c2])
                        sw = plsc.load_gather(ps[dp], [rows, c3])
                        dx = plsc.load_gather(pd[dp], [rows, zc])
                        dy = plsc.load_gather(pd[dp], [rows, c1])
                        dz = plsc.load_gather(pd[dp], [rows, c2])
                        ex = sx - dx
                        ey = sy - dy
                        ez = sz - dz
                        d2 = ex * ex + ey * ey + ez * ez
                        wo[dp][pl.ds(g * L, L)] = cmul * jnp.exp(nbw * d2) * sw
                        return c
                    lax.fori_loop(0, STEP // L, grp, 0)
                    e0 = row0 * CH + (m - 1) * STEP
                    pltpu.async_copy(wo[dp], w_hbm.at[pl.ds(e0, STEP)], sem_o[dp])
            return carry
        lax.fori_loop(0, Q, outer, 0)

    return k(es2d, ed2d, p2, prm)


def _sc_scatter(es2d, ed2d, wts, hcat, N):
    R = es2d.shape[0]
    RT = R // NS
    M = RT // NCH
    Q = (M + 2 + 3) // 4 + 1
    ACC_R = ((N + 1 + NS * CH - 1) // (NS * CH)) * NS * CH
    ZCH = ACC_R // (NS * CH)
    Nup = ((N + NS * 8 - 1) // (NS * 8)) * (NS * 8)
    ORT = Nup // NS

    mesh = plsc.VectorSubcoreMesh(core_axis_name="c", subcore_axis_name="s")

    @functools.partial(
        pl.kernel,
        out_type=jax.ShapeDtypeStruct((2 * Nup, 32), jnp.bfloat16),
        mesh=mesh,
        compiler_params=_SC_PARAMS,
        scratch_types=(
            [pltpu.VMEM((NCH, CH), jnp.int32) for _ in range(4)]
            + [pltpu.VMEM((NCH, CH), jnp.int32) for _ in range(4)]
            + [pltpu.VMEM((NCH, CH), jnp.int32) for _ in range(4)]
            + [pltpu.VMEM((STEP,), jnp.float32) for _ in range(4)]
            + [pltpu.VMEM((STEP, 32), jnp.bfloat16) for _ in range(2)]
            + [pltpu.VMEM_SHARED((ACC_R, 32), jnp.bfloat16)]
            + [pltpu.SemaphoreType.DMA for _ in range(8)]
        ),
    )
    def k(es_hbm, ed_hbm, w_hbm, h_hbm, out_hbm, *scr):
        si = scr[0:4]
        di = scr[4:8]
        s2 = scr[8:12]
        wv = scr[12:16]
        hs = scr[16:18]
        ho = scr[16:18]
        acc = scr[18]
        sem_i = scr[19:23]
        sem_g = scr[23:25]
        sem_s = scr[25:27]

        cid = lax.axis_index("c")
        sid = lax.axis_index("s")
        row0 = sid * RT
        cNv = jnp.full((L,), cid * N, jnp.int32)

        def _zrow(r, c):
            hs[0][r, pl.ds(0, 2 * L)] = jnp.zeros((2 * L,), jnp.bfloat16)
            return c
        lax.fori_loop(0, CH, _zrow, 0)
        for jz in range(ZCH):
            pltpu.sync_copy(hs[0].at[pl.ds(0, CH)],
                            acc.at[pl.ds(sid * ZCH * CH + jz * CH, CH)])
        plsc.subcore_barrier()

        def fire_idx(m, s):
            r0 = row0 + m * NCH
            pltpu.async_copy(es_hbm.at[pl.ds(r0, NCH)], si[s], sem_i[s])
            pltpu.async_copy(ed_hbm.at[pl.ds(r0, NCH)], di[s], sem_i[s])
            pltpu.async_copy(w_hbm.at[pl.ds(r0 * CH, STEP)], wv[s], sem_i[s])

        def wait_idx(m, s):
            r0 = row0 + m * NCH
            pltpu.make_async_copy(es_hbm.at[pl.ds(r0, NCH)], si[s], sem_i[s]).wait()
            pltpu.make_async_copy(ed_hbm.at[pl.ds(r0, NCH)], di[s], sem_i[s]).wait()
            pltpu.make_async_copy(w_hbm.at[pl.ds(r0 * CH, STEP)], wv[s], sem_i[s]).wait()

        fire_idx(0, 0)
        fire_idx(1, 1)

        def outer(q, carry):
            for b in range(4):
                m = q * 4 + b
                s = b
                sn = (b + 2) % 4
                sp = (b + 3) % 4
                d = b % 2
                dp = (b + 1) % 2

                @pl.when(jnp.logical_and(m >= 2, m <= M + 1))
                def _():
                    for j in range(NCH):
                        pltpu.make_async_copy(
                            ho[d].at[pl.ds(j * CH, CH)],
                            acc.at[di[sn].at[j]], sem_s[d]).wait()

                @pl.when(m + 2 <= M - 1)
                def _():
                    fire_idx(m + 2, sn)

                @pl.when(m <= M - 1)
                def _():
                    wait_idx(m, s)

                    def adds(t, c):
                        j = t // 8
                        col = (t % 8) * L
                        s2[s][j, pl.ds(col, L)] = si[s][j, pl.ds(col, L)] + cNv
                        return c
                    lax.fori_loop(0, NCH * 8, adds, 0)
                    for j in range(NCH):
                        pltpu.async_copy(h_hbm.at[s2[s].at[j]],
                                         hs[d].at[pl.ds(j * CH, CH)], sem_g[d])

                @pl.when(jnp.logical_and(m >= 1, m <= M))
                def _():
                    for j in range(NCH):
                        pltpu.make_async_copy(
                            h_hbm.at[s2[sp].at[j]],
                            hs[dp].at[pl.ds(j * CH, CH)], sem_g[dp]).wait()

                    def grp(g, c):
                        w16 = wv[sp][pl.ds(g * L, L)]
                        for e in range(L):
                            svf = jnp.full((L,), w16[e], jnp.float32)
                            svec = plsc.pack(svf, svf, format=plsc.PackFormat.INTERLEAVED)
                            row = g * L + e
                            hv = hs[dp][row, pl.ds(0, 2 * L)]
                            hs[dp][row, pl.ds(0, 2 * L)] = hv * svec
                        return c
                    lax.fori_loop(0, STEP // L, grp, 0)
                    for j in range(NCH):
                        pltpu.async_copy(ho[dp].at[pl.ds(j * CH, CH)],
                                         acc.at[di[sp].at[j]], sem_s[dp],
                                         add=True)
            return carry
        lax.fori_loop(0, Q, outer, 0)

        plsc.subcore_barrier()
        pltpu.sync_copy(acc.at[pl.ds(sid * ORT, ORT)],
                        out_hbm.at[pl.ds(cid * Nup + sid * ORT, ORT)])

    return k(es2d, ed2d, wts, hcat)


def kernel(x, grid, grid_weight, edge_src, edge_dst, W1, b1, W2, b2, baseweight):
    bsz, in_c, N = x.shape
    phy_dim = grid.shape[2]
    hid = W1.shape[0]
    out_c = W2.shape[0]
    K = edge_src.shape[2]
    E = bsz * N * K
    HH = hid // (2 * NC)

    NB = 1024
    Np = ((N + NB - 1) // NB) * NB
    nb = Np // NB

    x_p = jnp.pad(x[0], ((0, 0), (0, Np - N)))
    H = _fc1(x_p, W1, b1, nb)

    p2 = jnp.concatenate([grid[0], grid_weight[0][:, None]], axis=1)
    p2 = jnp.pad(p2, ((0, 16), (0, 12)))

    EPT = NC * NS * 8 * CH
    Ep = ((E + EPT - 1) // EPT) * EPT
    es = jnp.concatenate(
        [edge_src.reshape(-1), jnp.zeros((Ep - E,), jnp.int32)]).reshape(-1, CH)
    ed = jnp.concatenate(
        [edge_dst.reshape(-1), jnp.full((Ep - E,), N, jnp.int32)]).reshape(-1, CH)

    bw = baseweight.reshape(())
    cmul = jnp.sqrt((bw / math.pi) ** phy_dim)
    prm = jnp.concatenate([jnp.broadcast_to(-bw, (L,)),
                           jnp.broadcast_to(cmul, (L,))])

    wts = _sc_weights(es, ed, p2, prm)

    hcat = jnp.concatenate([H[:N, :hid // 2], H[:N, hid // 2:]],
                           axis=0).astype(jnp.bfloat16)
    msg = _sc_scatter(es, ed, wts, hcat, N)
    Nup = ((N + NS * 8 - 1) // (NS * 8)) * (NS * 8)

    a2 = jnp.concatenate([msg[:N], msg[Nup:Nup + N]],
                         axis=1).astype(jnp.float32)
    a2 = jnp.pad(a2, ((0, Np - N), (0, 0)))
    out = _fc2(a2, W2, b2, nb)
    return out[:, :N].reshape(bsz, out_c, N)
